# baseline (speedup 1.0000x reference)
import jax
import jax.numpy as jnp
from jax.experimental import pallas as pl
from jax.experimental.pallas import tpu as pltpu

_B, _N, _C = 4, 20000, 91
_NSEG = 112
_TILE = 128 * 128
_PADLEN = _NSEG * _TILE
_K = 100
_KPAD = 104

_BIG = 1 << 30


def _top100_kernel(p_ref, bx_ref, amt_ref, vals_ref, idx_ref, box_ref,
                   amtsel_ref, seg_ref):
    seg_i = jax.lax.broadcasted_iota(jnp.int32, (_NSEG, 1), 0)
    tile_fi = (jax.lax.broadcasted_iota(jnp.int32, (1, 128, 128), 1) * 128
               + jax.lax.broadcasted_iota(jnp.int32, (1, 128, 128), 2))

    for b in range(_B):
        m1 = jnp.max(p_ref[b], axis=1)
        seg_ref[b, :, 0:1] = jnp.max(m1, axis=1, keepdims=True)

    def body(k, carry):
        ns = []
        for b in range(_B):
            segmax = seg_ref[b, :, 0:1]
            v = jnp.max(segmax)
            s = jnp.min(jnp.where(segmax == v, seg_i, _BIG))
            tile = p_ref[b, pl.ds(s, 1), :, :]
            fin = jnp.min(jnp.where(tile == v, tile_fi, _BIG))
            f = s * _TILE + fin
            n = f // _C
            ns.append(n)
            vals_ref[b, pl.ds(k, 1), :] = jnp.full((1, 128), v,
                                                   dtype=jnp.float32)
            idx_ref[b, pl.ds(k, 1), :] = jnp.full((1, 128), f,
                                                  dtype=jnp.int32)
            new_tile = jnp.where(tile_fi == fin, -1.0, tile)
            p_ref[b, pl.ds(s, 1), :, :] = new_tile
            m = jnp.max(new_tile)
            seg_ref[b, pl.ds(s, 1), 0:1] = jnp.full((1, 1), m,
                                                    dtype=jnp.float32)
            box_ref[b, pl.ds(k, 1), :] = bx_ref[b, pl.ds(n // 8, 1), :]
        n0 = ns[0]
        for b in range(_B):
            amtsel_ref[b, pl.ds(k, 1), :] = amt_ref[b, pl.ds(n0 // 8, 1), :]
        return carry

    jax.lax.fori_loop(0, _K, body, 0)


def _top3_kernel(w_ref, bx_ref, vals_ref, box_ref):
    fi = (jax.lax.broadcasted_iota(jnp.int32, (160, 128), 0) * 128
          + jax.lax.broadcasted_iota(jnp.int32, (160, 128), 1))

    def body(k, carry):
        arr = w_ref[0, :, :]
        v = jnp.max(arr)
        f = jnp.min(jnp.where(arr == v, fi, _BIG))
        vals_ref[0, pl.ds(k, 1), :] = jnp.full((1, 128), v, dtype=jnp.float32)
        box_ref[0, pl.ds(k, 1), :] = bx_ref[0, pl.ds(f, 1), :]
        w_ref[0, :, :] = jnp.where(fi == f, -jnp.inf, arr)
        return carry

    jax.lax.fori_loop(0, 3, body, 0)


@jax.jit
def kernel(pred_logits, pred_boxes, target_sizes, amount_score,
           service_pred_logits, hs_output_weights, enc_output_weights):
    B, N, C = pred_logits.shape
    nsac = service_pred_logits.shape[1]

    prob = jax.nn.sigmoid(pred_logits).reshape(B, N * C)
    prob = jnp.pad(prob, ((0, 0), (0, _PADLEN - N * C)), constant_values=-1.0)
    prob = prob.reshape(B, _NSEG, 128, 128)

    cx, cy, w, h = (pred_boxes[..., 0], pred_boxes[..., 1],
                    pred_boxes[..., 2], pred_boxes[..., 3])
    boxes_xyxy = jnp.stack([cx - 0.5 * w, cy - 0.5 * h,
                            cx + 0.5 * w, cy + 0.5 * h], axis=-1)
    bx32 = boxes_xyxy.reshape(B, N // 8, 32)
    amt32 = amount_score.reshape(B, N // 8, 32)

    vals, idx, boxsel, amtsel = pl.pallas_call(
        _top100_kernel,
        grid=(1,),
        in_specs=[
            pl.BlockSpec((B, _NSEG, 128, 128), lambda i: (0, 0, 0, 0)),
            pl.BlockSpec((B, N // 8, 32), lambda i: (0, 0, 0)),
            pl.BlockSpec((B, N // 8, 32), lambda i: (0, 0, 0)),
        ],
        out_specs=[
            pl.BlockSpec((B, _KPAD, 128), lambda i: (0, 0, 0)),
            pl.BlockSpec((B, _KPAD, 128), lambda i: (0, 0, 0)),
            pl.BlockSpec((B, _KPAD, 32), lambda i: (0, 0, 0)),
            pl.BlockSpec((B, _KPAD, 32), lambda i: (0, 0, 0)),
        ],
        out_shape=[
            jax.ShapeDtypeStruct((B, _KPAD, 128), jnp.float32),
            jax.ShapeDtypeStruct((B, _KPAD, 128), jnp.int32),
            jax.ShapeDtypeStruct((B, _KPAD, 32), jnp.float32),
            jax.ShapeDtypeStruct((B, _KPAD, 32), jnp.float32),
        ],
        scratch_shapes=[
            pltpu.VMEM((B, _NSEG, 128), jnp.float32),
        ],
    )(prob, bx32, amt32)

    scores = vals[:, :_K, 0]
    flat_idx = idx[:, :_K, 0]
    labels = flat_idx % C
    topk_boxes = flat_idx // C

    img_h = target_sizes[:, 0]
    img_w = target_sizes[:, 1]
    scale_fct = jnp.stack([img_w, img_h, img_w, img_h], axis=1)

    oh = jax.nn.one_hot(topk_boxes % 8, 8, dtype=jnp.float32)
    boxes = (boxsel[:, :_K, :].reshape(B, _K, 8, 4)
             * oh[..., None]).sum(axis=2)
    boxes = boxes * scale_fct[:, None, :]
    oh0 = jax.nn.one_hot(topk_boxes[0] % 8, 8, dtype=jnp.float32)
    amount_score_sel = (amtsel[:, :_K, :].reshape(B, _K, 8, 4)
                        * oh0[None, :, :, None]).sum(axis=2)

    wrows = jnp.stack([hs_output_weights, enc_output_weights])[:, :, 1:, :]
    ncls = nsac - 1
    nrows = 2 * B * ncls
    wrows = wrows.reshape(nrows, N)
    wrows = jnp.pad(wrows, ((0, 0), (0, 160 * 128 - N)),
                    constant_values=-jnp.inf).reshape(nrows, 160, 128)

    vals3, box3 = pl.pallas_call(
        _top3_kernel,
        grid=(nrows,),
        in_specs=[
            pl.BlockSpec((1, 160, 128), lambda i: (i, 0, 0)),
            pl.BlockSpec((1, N, 4), lambda i: ((i // ncls) % _B, 0, 0)),
        ],
        out_specs=[
            pl.BlockSpec((1, 8, 128), lambda i: (i, 0, 0)),
            pl.BlockSpec((1, 8, 4), lambda i: (i, 0, 0)),
        ],
        out_shape=[
            jax.ShapeDtypeStruct((nrows, 8, 128), jnp.float32),
            jax.ShapeDtypeStruct((nrows, 8, 4), jnp.float32),
        ],
    )(wrows, boxes_xyxy)

    attn_vals = vals3[:, :3, 0].reshape(2, B, ncls, 3)
    attn_box = (box3[:, :3, :].reshape(2, B, ncls, 3, 4)
                * scale_fct[:, None, None, :])
    hs_attn_values, enc_attn_values = attn_vals[0], attn_vals[1]
    hs_attn_bbox, enc_attn_bbox = attn_box[0], attn_box[1]

    return (scores, labels, boxes, amount_score_sel,
            hs_attn_values, hs_attn_bbox, enc_attn_values, enc_attn_bbox)


# speedup vs baseline: 2.4136x; 2.4136x over previous
import jax
import jax.numpy as jnp
from jax.experimental import pallas as pl
from jax.experimental.pallas import tpu as pltpu

_B, _N, _C = 4, 20000, 91
_NSEG = 100
_SROWS = 200
_TILE = _SROWS * _C
_K = 100
_KPAD = 104

_BIG = 1 << 30


def _top100_kernel(p_ref, bx_ref, amt_ref, vals_ref, idx_ref, box_ref,
                   amtsel_ref, seg_ref):
    seg_i = jax.lax.broadcasted_iota(jnp.int32, (_NSEG, 1), 0)
    tile_fi = (jax.lax.broadcasted_iota(jnp.int32, (_SROWS, _C), 0) * _C
               + jax.lax.broadcasted_iota(jnp.int32, (_SROWS, _C), 1))

    for b in range(_B):
        for s in range(_NSEG):
            m = jnp.max(p_ref[b, s * _SROWS:(s + 1) * _SROWS, :])
            seg_ref[b, s:s + 1, 0:1] = jnp.full((1, 1), m,
                                                dtype=jnp.float32)

    def body(k, carry):
        ns = []
        for b in range(_B):
            segmax = seg_ref[b, 0:_NSEG, 0:1]
            v = jnp.max(segmax)
            s = jnp.min(jnp.where(segmax == v, seg_i, _BIG))
            tile = p_ref[b, pl.ds(s * _SROWS, _SROWS), :]
            fin = jnp.min(jnp.where(tile == v, tile_fi, _BIG))
            f = s * _TILE + fin
            n = f // _C
            ns.append(n)
            vals_ref[b, pl.ds(k, 1), :] = jnp.full((1, 128), v,
                                                   dtype=jnp.float32)
            idx_ref[b, pl.ds(k, 1), :] = jnp.full((1, 128), f,
                                                  dtype=jnp.int32)
            new_tile = jnp.where(tile_fi == fin, -1.0, tile)
            p_ref[b, pl.ds(s * _SROWS, _SROWS), :] = new_tile
            m = jnp.max(new_tile)
            seg_ref[b, pl.ds(s, 1), 0:1] = jnp.full((1, 1), m,
                                                    dtype=jnp.float32)
            box_ref[b, pl.ds(k, 1), :] = bx_ref[b, pl.ds(n // 8, 1), :]
        n0 = ns[0]
        for b in range(_B):
            amtsel_ref[b, pl.ds(k, 1), :] = amt_ref[b, pl.ds(n0 // 8, 1), :]
        return carry

    jax.lax.fori_loop(0, _K, body, 0)


def _top3_kernel(w_ref, bx_ref, vals_ref, box_ref):
    fi = (jax.lax.broadcasted_iota(jnp.int32, (160, 128), 0) * 128
          + jax.lax.broadcasted_iota(jnp.int32, (160, 128), 1))

    def body(k, carry):
        arr = w_ref[0, :, :]
        v = jnp.max(arr)
        f = jnp.min(jnp.where(arr == v, fi, _BIG))
        vals_ref[0, pl.ds(k, 1), :] = jnp.full((1, 128), v, dtype=jnp.float32)
        box_ref[0, pl.ds(k, 1), :] = bx_ref[0, pl.ds(f, 1), :]
        w_ref[0, :, :] = jnp.where(fi == f, -jnp.inf, arr)
        return carry

    jax.lax.fori_loop(0, 3, body, 0)


@jax.jit
def kernel(pred_logits, pred_boxes, target_sizes, amount_score,
           service_pred_logits, hs_output_weights, enc_output_weights):
    B, N, C = pred_logits.shape
    nsac = service_pred_logits.shape[1]

    prob = jax.nn.sigmoid(pred_logits)

    cx, cy, w, h = (pred_boxes[..., 0], pred_boxes[..., 1],
                    pred_boxes[..., 2], pred_boxes[..., 3])
    boxes_xyxy = jnp.stack([cx - 0.5 * w, cy - 0.5 * h,
                            cx + 0.5 * w, cy + 0.5 * h], axis=-1)
    bx32 = boxes_xyxy.reshape(B, N // 8, 32)
    amt32 = amount_score.reshape(B, N // 8, 32)

    vals, idx, boxsel, amtsel = pl.pallas_call(
        _top100_kernel,
        grid=(1,),
        in_specs=[
            pl.BlockSpec((B, N, C), lambda i: (0, 0, 0)),
            pl.BlockSpec((B, N // 8, 32), lambda i: (0, 0, 0)),
            pl.BlockSpec((B, N // 8, 32), lambda i: (0, 0, 0)),
        ],
        out_specs=[
            pl.BlockSpec((B, _KPAD, 128), lambda i: (0, 0, 0)),
            pl.BlockSpec((B, _KPAD, 128), lambda i: (0, 0, 0)),
            pl.BlockSpec((B, _KPAD, 32), lambda i: (0, 0, 0)),
            pl.BlockSpec((B, _KPAD, 32), lambda i: (0, 0, 0)),
        ],
        out_shape=[
            jax.ShapeDtypeStruct((B, _KPAD, 128), jnp.float32),
            jax.ShapeDtypeStruct((B, _KPAD, 128), jnp.int32),
            jax.ShapeDtypeStruct((B, _KPAD, 32), jnp.float32),
            jax.ShapeDtypeStruct((B, _KPAD, 32), jnp.float32),
        ],
        scratch_shapes=[
            pltpu.VMEM((B, 104, 128), jnp.float32),
        ],
    )(prob, bx32, amt32)

    scores = vals[:, :_K, 0]
    flat_idx = idx[:, :_K, 0]
    labels = flat_idx % C
    topk_boxes = flat_idx // C

    img_h = target_sizes[:, 0]
    img_w = target_sizes[:, 1]
    scale_fct = jnp.stack([img_w, img_h, img_w, img_h], axis=1)

    oh = jax.nn.one_hot(topk_boxes % 8, 8, dtype=jnp.float32)
    boxes = (boxsel[:, :_K, :].reshape(B, _K, 8, 4)
             * oh[..., None]).sum(axis=2)
    boxes = boxes * scale_fct[:, None, :]
    oh0 = jax.nn.one_hot(topk_boxes[0] % 8, 8, dtype=jnp.float32)
    amount_score_sel = (amtsel[:, :_K, :].reshape(B, _K, 8, 4)
                        * oh0[None, :, :, None]).sum(axis=2)

    wrows = jnp.stack([hs_output_weights, enc_output_weights])[:, :, 1:, :]
    ncls = nsac - 1
    nrows = 2 * B * ncls
    wrows = wrows.reshape(nrows, N)
    wrows = jnp.pad(wrows, ((0, 0), (0, 160 * 128 - N)),
                    constant_values=-jnp.inf).reshape(nrows, 160, 128)

    vals3, box3 = pl.pallas_call(
        _top3_kernel,
        grid=(nrows,),
        in_specs=[
            pl.BlockSpec((1, 160, 128), lambda i: (i, 0, 0)),
            pl.BlockSpec((1, N, 4), lambda i: ((i // ncls) % _B, 0, 0)),
        ],
        out_specs=[
            pl.BlockSpec((1, 8, 128), lambda i: (i, 0, 0)),
            pl.BlockSpec((1, 8, 4), lambda i: (i, 0, 0)),
        ],
        out_shape=[
            jax.ShapeDtypeStruct((nrows, 8, 128), jnp.float32),
            jax.ShapeDtypeStruct((nrows, 8, 4), jnp.float32),
        ],
    )(wrows, boxes_xyxy)

    attn_vals = vals3[:, :3, 0].reshape(2, B, ncls, 3)
    attn_box = (box3[:, :3, :].reshape(2, B, ncls, 3, 4)
                * scale_fct[:, None, None, :])
    hs_attn_values, enc_attn_values = attn_vals[0], attn_vals[1]
    hs_attn_bbox, enc_attn_bbox = attn_box[0], attn_box[1]

    return (scores, labels, boxes, amount_score_sel,
            hs_attn_values, hs_attn_bbox, enc_attn_values, enc_attn_bbox)
